# baseline (speedup 1.0000x reference)
import jax
import jax.numpy as jnp
from jax import lax
from jax.experimental import pallas as pl
from jax.experimental.pallas import tpu as pltpu
from jax.experimental.pallas import tpu_sc as plsc
import numpy as np

N_PTS = 131072
N_LEVELS = 16
BASE_RES = 16
PLS = 1.47
N_SPLASH = 4
T = 2 ** 17
P2 = np.int32(np.uint32(2654435761).astype(np.int64) - (1 << 32))
P3 = np.int32(805459861)
DIRS = np.array([[0.0, 0.0, 0.0],
                 [0.57735, 0.57735, 0.57735],
                 [-0.57735, 0.57735, -0.57735],
                 [0.57735, -0.57735, -0.57735]], dtype=np.float32)
RES = np.array([BASE_RES * (PLS ** l) for l in range(N_LEVELS)], dtype=np.float32)

NC, NS = 2, 16
LV_PER_CORE = N_LEVELS // NC
PB = N_PTS // NS
CH = 256
NCHUNK = PB // CH
LOOK = N_SPLASH * 8
GB = LOOK * CH
NLC = 13


def _prep_body(log_std_ref, logits_ref, dirs_ref, res_ref, gmm_ref, lvl13_ref):
    ls = log_std_ref[0, :]
    std = jnp.exp(ls)
    lg = logits_ref[0, :]
    m = jnp.max(lg)
    e = jnp.exp(lg - m)
    sw = e / jnp.sum(e)
    gmm_ref[0, :] = jnp.concatenate([std, sw])
    offs = dirs_ref[0, :][:, None] * std[None, :]
    row_res = res_ref[0, :][None, :]
    lvl13_ref[...] = jnp.concatenate([offs, row_res], axis=0)


def _sw_body(logits_ref, swb_ref):
    lg = logits_ref[0, :]
    m = jnp.max(lg)
    e = jnp.exp(lg - m)
    sw = e / jnp.sum(e)
    swb_ref[...] = jnp.broadcast_to(sw[:, None], (N_SPLASH, 16))


def _encode_body(coords_ref, tpk_ref, lvlc_ref, swb_ref,
                 enc0_ref, enc1_ref,
                 cbuf, swb_v, offs_lv,
                 idx0, idx1, w0, w1, g0, g1,
                 e0, e1, sem0, sem1):
    c = lax.axis_index("c")
    s = lax.axis_index("s")
    base = s * PB

    pltpu.sync_copy(coords_ref.at[pl.ds(base, PB)], cbuf.at[pl.ds(0, PB)])
    pltpu.sync_copy(coords_ref.at[pl.ds(N_PTS + base, PB)],
                    cbuf.at[pl.ds(PB, PB)])
    pltpu.sync_copy(coords_ref.at[pl.ds(2 * N_PTS + base, PB)],
                    cbuf.at[pl.ds(2 * PB, PB)])
    pltpu.sync_copy(swb_ref, swb_v)

    def produce(lvl_off, j, idx_b, w_b):
        res = offs_lv[pl.ds(12 * 16, 16)]

        mask_c = jnp.int32(T - 1)
        mag = jnp.float32(12582912.0)

        def prod_i(i, c2):
            po = j * CH + i * 16
            x = cbuf[pl.ds(po, 16)]
            y = cbuf[pl.ds(PB + po, 16)]
            z = cbuf[pl.ds(2 * PB + po, 16)]
            for s_ in range(N_SPLASH):
                qx = (x + offs_lv[pl.ds((s_ * 3 + 0) * 16, 16)]) * res
                qy = (y + offs_lv[pl.ds((s_ * 3 + 1) * 16, 16)]) * res
                qz = (z + offs_lv[pl.ds((s_ * 3 + 2) * 16, 16)]) * res
                fx = ((qx - 0.5) + mag) - mag
                fy = ((qy - 0.5) + mag) - mag
                fz = ((qz - 0.5) + mag) - mag
                tx = qx - fx
                ty = qy - fy
                tz = qz - fz
                ix = fx.astype(jnp.int32)
                iy = fy.astype(jnp.int32)
                iz = fz.astype(jnp.int32)
                hx = [ix & mask_c, (ix + 1) & mask_c]
                hy0 = iy * P2
                hz0 = iz * P3
                hy1 = hy0 + P2
                hz1 = hz0 + P3
                eyz = [[((hy0 ^ hz0) & mask_c) | lvl_off,
                        ((hy0 ^ hz1) & mask_c) | lvl_off],
                       [((hy1 ^ hz0) & mask_c) | lvl_off,
                        ((hy1 ^ hz1) & mask_c) | lvl_off]]
                sws = swb_v[s_, :]
                wx1 = tx * sws
                wx0 = sws - wx1
                wy = [1.0 - ty, ty]
                wz = [1.0 - tz, tz]
                wxy = [[wx0 * wy[0], wx0 * wy[1]], [wx1 * wy[0], wx1 * wy[1]]]
                for dx in range(2):
                    for dy in range(2):
                        for dz in range(2):
                            k = s_ * 8 + dx * 4 + dy * 2 + dz
                            off = i * (LOOK * 16) + k * 16
                            idx_b[pl.ds(off, 16)] = hx[dx] ^ eyz[dy][dz]
                            w_b[pl.ds(off, 16)] = wxy[dx][dy] * wz[dz]
            return c2

        lax.fori_loop(0, CH // 16, prod_i, 0)

    def consume(j, w_b, g_b):
        def cons_i(i, c2):
            a0 = jnp.zeros((16,), jnp.float32)
            a1 = jnp.zeros((16,), jnp.float32)
            for k in range(LOOK):
                off = i * (LOOK * 16) + k * 16
                w = w_b[pl.ds(off, 16)]
                g = g_b[pl.ds(off, 16)]
                f0 = lax.bitcast_convert_type(g << 16, jnp.float32)
                f1 = lax.bitcast_convert_type(g & jnp.int32(-65536), jnp.float32)
                a0 = a0 + w * f0
                a1 = a1 + w * f1
            po = j * CH + i * 16
            e0[pl.ds(po, 16)] = a0
            e1[pl.ds(po, 16)] = a1
            return c2

        lax.fori_loop(0, CH // 16, cons_i, 0)

    def level_body(l_local, carry):
        lv_g = l_local * NC + c
        lvl_off = lv_g * T
        pltpu.sync_copy(lvlc_ref.at[pl.ds(lv_g * NLC * 16, NLC * 16)], offs_lv)
        produce(lvl_off, 0, idx0, w0)
        pltpu.async_copy(tpk_ref.at[idx0], g0, sem0)
        bufs = ((idx0, w0, g0, sem0), (idx1, w1, g1, sem1))

        def chunk_pair(j2, carry2):
            for b in range(2):
                j = j2 * 2 + b
                idx_b, w_b, g_b, sem = bufs[b]
                idx_n, w_n, g_n, sem_n = bufs[1 - b]

                @pl.when(j < NCHUNK - 1)
                def _():
                    produce(lvl_off, j + 1, idx_n, w_n)
                    pltpu.async_copy(tpk_ref.at[idx_n], g_n, sem_n)

                pltpu.make_async_copy(tpk_ref.at[idx_b], g_b, sem).wait()
                consume(j, w_b, g_b)
            return carry2

        lax.fori_loop(0, NCHUNK // 2, chunk_pair, 0)
        pltpu.sync_copy(e0, enc0_ref.at[pl.ds(lv_g * N_PTS + base, PB)])
        pltpu.sync_copy(e1, enc1_ref.at[pl.ds(lv_g * N_PTS + base, PB)])
        return carry

    lax.fori_loop(0, LV_PER_CORE, level_body, 0)


def _mlp_body(enc0_ref, enc1_ref, w0e_ref, w0o_ref, w1_ref, w2_ref, out_ref):
    e0 = enc0_ref[...]
    e1 = enc1_ref[...]
    h = jax.nn.relu(
        jax.lax.dot_general(e0, w0e_ref[...], (((0,), (0,)), ((), ())),
                            preferred_element_type=jnp.float32)
        + jax.lax.dot_general(e1, w0o_ref[...], (((0,), (0,)), ((), ())),
                              preferred_element_type=jnp.float32))
    h = jax.nn.relu(jnp.dot(h, w1_ref[...], preferred_element_type=jnp.float32))
    out_ref[...] = jnp.dot(h, w2_ref[...], preferred_element_type=jnp.float32)


def kernel(coords, table, log_std, splash_logits, W0, W1, W2):
    coords_f = coords.T.reshape(3 * N_PTS)
    tb = lax.bitcast_convert_type(table.astype(jnp.bfloat16),
                                  jnp.uint16).astype(jnp.uint32)
    tpk = ((tb[..., 1] << 16) | tb[..., 0]).astype(jnp.int32).reshape(N_LEVELS * T)
    w0e = W0[0::2]
    w0o = W0[1::2]

    gmm2, lvl13 = pl.pallas_call(
        _prep_body,
        out_shape=(jax.ShapeDtypeStruct((1, N_LEVELS + N_SPLASH), jnp.float32),
                   jax.ShapeDtypeStruct((NLC, N_LEVELS), jnp.float32)),
    )(log_std.reshape(1, N_LEVELS), splash_logits.reshape(1, N_SPLASH),
      jnp.asarray(DIRS.reshape(1, 12)), jnp.asarray(RES.reshape(1, N_LEVELS)))
    gmm = gmm2[0]
    lvlc = jnp.broadcast_to(lvl13.T[:, :, None], (N_LEVELS, NLC, 16))

    swb = pl.pallas_call(
        _sw_body,
        out_shape=jax.ShapeDtypeStruct((N_SPLASH, 16), jnp.float32),
    )(splash_logits.reshape(1, N_SPLASH))

    mesh = plsc.VectorSubcoreMesh(core_axis_name="c", subcore_axis_name="s")
    enc0f, enc1f = pl.kernel(
        _encode_body,
        out_type=(jax.ShapeDtypeStruct((N_LEVELS * N_PTS,), jnp.float32),
                  jax.ShapeDtypeStruct((N_LEVELS * N_PTS,), jnp.float32)),
        mesh=mesh,
        scratch_types=[
            pltpu.VMEM((3 * PB,), jnp.float32),
            pltpu.VMEM((N_SPLASH, 16), jnp.float32),
            pltpu.VMEM((NLC * 16,), jnp.float32),
            pltpu.VMEM((GB,), jnp.int32),
            pltpu.VMEM((GB,), jnp.int32),
            pltpu.VMEM((GB,), jnp.float32),
            pltpu.VMEM((GB,), jnp.float32),
            pltpu.VMEM((GB,), jnp.int32),
            pltpu.VMEM((GB,), jnp.int32),
            pltpu.VMEM((PB,), jnp.float32),
            pltpu.VMEM((PB,), jnp.float32),
            pltpu.SemaphoreType.DMA,
            pltpu.SemaphoreType.DMA,
        ],
    )(coords_f, tpk, lvlc.reshape(N_LEVELS * NLC * 16), swb)
    enc0 = enc0f.reshape(N_LEVELS, N_PTS)
    enc1 = enc1f.reshape(N_LEVELS, N_PTS)

    blk = 2048
    out = pl.pallas_call(
        _mlp_body,
        grid=(N_PTS // blk,),
        in_specs=[
            pl.BlockSpec((N_LEVELS, blk), lambda i: (0, i)),
            pl.BlockSpec((N_LEVELS, blk), lambda i: (0, i)),
            pl.BlockSpec((N_LEVELS, 64), lambda i: (0, 0)),
            pl.BlockSpec((N_LEVELS, 64), lambda i: (0, 0)),
            pl.BlockSpec((64, 64), lambda i: (0, 0)),
            pl.BlockSpec((64, 3), lambda i: (0, 0)),
        ],
        out_specs=pl.BlockSpec((blk, 3), lambda i: (i, 0)),
        out_shape=jax.ShapeDtypeStruct((N_PTS, 3), jnp.float32),
    )(enc0, enc1, w0e, w0o, W1, W2)

    return (out, gmm)


# speedup vs baseline: 1.0154x; 1.0154x over previous
import jax
import jax.numpy as jnp
from jax import lax
from jax.experimental import pallas as pl
from jax.experimental.pallas import tpu as pltpu
from jax.experimental.pallas import tpu_sc as plsc
import numpy as np

N_PTS = 131072
N_LEVELS = 16
BASE_RES = 16
PLS = 1.47
N_SPLASH = 4
T = 2 ** 17
P2 = np.int32(np.uint32(2654435761).astype(np.int64) - (1 << 32))
P3 = np.int32(805459861)
DIRS = np.array([[0.0, 0.0, 0.0],
                 [0.57735, 0.57735, 0.57735],
                 [-0.57735, 0.57735, -0.57735],
                 [0.57735, -0.57735, -0.57735]], dtype=np.float32)
RES = np.array([BASE_RES * (PLS ** l) for l in range(N_LEVELS)], dtype=np.float32)

NC, NS = 2, 16
LV_PER_CORE = N_LEVELS // NC
PB = N_PTS // NS
CH = 128
NBUF = 4
NCHUNK = PB // CH
LOOK = N_SPLASH * 8
GB = LOOK * CH
NLC = 13


def _prep_body(log_std_ref, logits_ref, dirs_ref, res_ref, gmm_ref, lvl13_ref):
    ls = log_std_ref[0, :]
    std = jnp.exp(ls)
    lg = logits_ref[0, :]
    m = jnp.max(lg)
    e = jnp.exp(lg - m)
    sw = e / jnp.sum(e)
    gmm_ref[0, :] = jnp.concatenate([std, sw])
    offs = dirs_ref[0, :][:, None] * std[None, :]
    row_res = res_ref[0, :][None, :]
    lvl13_ref[...] = jnp.concatenate([offs, row_res], axis=0)


def _sw_body(logits_ref, swb_ref):
    lg = logits_ref[0, :]
    m = jnp.max(lg)
    e = jnp.exp(lg - m)
    sw = e / jnp.sum(e)
    swb_ref[...] = jnp.broadcast_to(sw[:, None], (N_SPLASH, 16))


def _encode_body(coords_ref, tpk_ref, lvlc_ref, swb_ref,
                 enc0_ref, enc1_ref,
                 cbuf, swb_v, offs_lv,
                 idx0, idx1, idx2, idx3, w0, w1, w2, w3,
                 g0, g1, g2, g3,
                 e0, e1, sem0, sem1, sem2, sem3):
    c = lax.axis_index("c")
    s = lax.axis_index("s")
    base = s * PB

    pltpu.sync_copy(coords_ref.at[pl.ds(base, PB)], cbuf.at[pl.ds(0, PB)])
    pltpu.sync_copy(coords_ref.at[pl.ds(N_PTS + base, PB)],
                    cbuf.at[pl.ds(PB, PB)])
    pltpu.sync_copy(coords_ref.at[pl.ds(2 * N_PTS + base, PB)],
                    cbuf.at[pl.ds(2 * PB, PB)])
    pltpu.sync_copy(swb_ref, swb_v)

    def produce(lvl_off, j, idx_b, w_b):
        res = offs_lv[pl.ds(12 * 16, 16)]

        mask_c = jnp.int32(T - 1)
        mag = jnp.float32(12582912.0)

        def prod_i(i, c2):
            po = j * CH + i * 16
            x = cbuf[pl.ds(po, 16)]
            y = cbuf[pl.ds(PB + po, 16)]
            z = cbuf[pl.ds(2 * PB + po, 16)]
            for s_ in range(N_SPLASH):
                qx = (x + offs_lv[pl.ds((s_ * 3 + 0) * 16, 16)]) * res
                qy = (y + offs_lv[pl.ds((s_ * 3 + 1) * 16, 16)]) * res
                qz = (z + offs_lv[pl.ds((s_ * 3 + 2) * 16, 16)]) * res
                fx = ((qx - 0.5) + mag) - mag
                fy = ((qy - 0.5) + mag) - mag
                fz = ((qz - 0.5) + mag) - mag
                tx = qx - fx
                ty = qy - fy
                tz = qz - fz
                ix = fx.astype(jnp.int32)
                iy = fy.astype(jnp.int32)
                iz = fz.astype(jnp.int32)
                hx = [ix & mask_c, (ix + 1) & mask_c]
                hy0 = iy * P2
                hz0 = iz * P3
                hy1 = hy0 + P2
                hz1 = hz0 + P3
                eyz = [[((hy0 ^ hz0) & mask_c) | lvl_off,
                        ((hy0 ^ hz1) & mask_c) | lvl_off],
                       [((hy1 ^ hz0) & mask_c) | lvl_off,
                        ((hy1 ^ hz1) & mask_c) | lvl_off]]
                sws = swb_v[s_, :]
                wx1 = tx * sws
                wx0 = sws - wx1
                wy = [1.0 - ty, ty]
                wz = [1.0 - tz, tz]
                wxy = [[wx0 * wy[0], wx0 * wy[1]], [wx1 * wy[0], wx1 * wy[1]]]
                for dx in range(2):
                    for dy in range(2):
                        for dz in range(2):
                            k = s_ * 8 + dx * 4 + dy * 2 + dz
                            off = i * (LOOK * 16) + k * 16
                            idx_b[pl.ds(off, 16)] = hx[dx] ^ eyz[dy][dz]
                            w_b[pl.ds(off, 16)] = wxy[dx][dy] * wz[dz]
            return c2

        lax.fori_loop(0, CH // 16, prod_i, 0)

    def consume(j, w_b, g_b):
        def cons_i(i, c2):
            a0 = jnp.zeros((16,), jnp.float32)
            a1 = jnp.zeros((16,), jnp.float32)
            for k in range(LOOK):
                off = i * (LOOK * 16) + k * 16
                w = w_b[pl.ds(off, 16)]
                g = g_b[pl.ds(off, 16)]
                f0 = lax.bitcast_convert_type(g << 16, jnp.float32)
                f1 = lax.bitcast_convert_type(g & jnp.int32(-65536), jnp.float32)
                a0 = a0 + w * f0
                a1 = a1 + w * f1
            po = j * CH + i * 16
            e0[pl.ds(po, 16)] = a0
            e1[pl.ds(po, 16)] = a1
            return c2

        lax.fori_loop(0, CH // 16, cons_i, 0)

    def level_body(l_local, carry):
        lv_g = l_local * NC + c
        lvl_off = lv_g * T
        pltpu.sync_copy(lvlc_ref.at[pl.ds(lv_g * NLC * 16, NLC * 16)], offs_lv)
        bufs = ((idx0, w0, g0, sem0), (idx1, w1, g1, sem1),
                (idx2, w2, g2, sem2), (idx3, w3, g3, sem3))
        for b in range(NBUF - 1):
            produce(lvl_off, b, bufs[b][0], bufs[b][1])
            pltpu.async_copy(tpk_ref.at[bufs[b][0]], bufs[b][2], bufs[b][3])

        def chunk_grp(j2, carry2):
            for b in range(NBUF):
                j = j2 * NBUF + b
                idx_b, w_b, g_b, sem = bufs[b]
                idx_n, w_n, g_n, sem_n = bufs[(b + NBUF - 1) % NBUF]

                @pl.when(j + NBUF - 1 < NCHUNK)
                def _():
                    produce(lvl_off, j + NBUF - 1, idx_n, w_n)
                    pltpu.async_copy(tpk_ref.at[idx_n], g_n, sem_n)

                pltpu.make_async_copy(tpk_ref.at[idx_b], g_b, sem).wait()
                consume(j, w_b, g_b)
            return carry2

        lax.fori_loop(0, NCHUNK // NBUF, chunk_grp, 0)
        pltpu.sync_copy(e0, enc0_ref.at[pl.ds(lv_g * N_PTS + base, PB)])
        pltpu.sync_copy(e1, enc1_ref.at[pl.ds(lv_g * N_PTS + base, PB)])
        return carry

    lax.fori_loop(0, LV_PER_CORE, level_body, 0)


def _mlp_body(enc0_ref, enc1_ref, w0e_ref, w0o_ref, w1_ref, w2_ref, out_ref):
    e0 = enc0_ref[...]
    e1 = enc1_ref[...]
    h = jax.nn.relu(
        jax.lax.dot_general(e0, w0e_ref[...], (((0,), (0,)), ((), ())),
                            preferred_element_type=jnp.float32)
        + jax.lax.dot_general(e1, w0o_ref[...], (((0,), (0,)), ((), ())),
                              preferred_element_type=jnp.float32))
    h = jax.nn.relu(jnp.dot(h, w1_ref[...], preferred_element_type=jnp.float32))
    out_ref[...] = jnp.dot(h, w2_ref[...], preferred_element_type=jnp.float32)


def kernel(coords, table, log_std, splash_logits, W0, W1, W2):
    coords_f = coords.T.reshape(3 * N_PTS)
    tb = lax.bitcast_convert_type(table.astype(jnp.bfloat16),
                                  jnp.uint16).astype(jnp.uint32)
    tpk = ((tb[..., 1] << 16) | tb[..., 0]).astype(jnp.int32).reshape(N_LEVELS * T)
    w0e = W0[0::2]
    w0o = W0[1::2]

    gmm2, lvl13 = pl.pallas_call(
        _prep_body,
        out_shape=(jax.ShapeDtypeStruct((1, N_LEVELS + N_SPLASH), jnp.float32),
                   jax.ShapeDtypeStruct((NLC, N_LEVELS), jnp.float32)),
    )(log_std.reshape(1, N_LEVELS), splash_logits.reshape(1, N_SPLASH),
      jnp.asarray(DIRS.reshape(1, 12)), jnp.asarray(RES.reshape(1, N_LEVELS)))
    gmm = gmm2[0]
    lvlc = jnp.broadcast_to(lvl13.T[:, :, None], (N_LEVELS, NLC, 16))

    swb = pl.pallas_call(
        _sw_body,
        out_shape=jax.ShapeDtypeStruct((N_SPLASH, 16), jnp.float32),
    )(splash_logits.reshape(1, N_SPLASH))

    mesh = plsc.VectorSubcoreMesh(core_axis_name="c", subcore_axis_name="s")
    enc0f, enc1f = pl.kernel(
        _encode_body,
        out_type=(jax.ShapeDtypeStruct((N_LEVELS * N_PTS,), jnp.float32),
                  jax.ShapeDtypeStruct((N_LEVELS * N_PTS,), jnp.float32)),
        mesh=mesh,
        scratch_types=[
            pltpu.VMEM((3 * PB,), jnp.float32),
            pltpu.VMEM((N_SPLASH, 16), jnp.float32),
            pltpu.VMEM((NLC * 16,), jnp.float32),
            pltpu.VMEM((GB,), jnp.int32),
            pltpu.VMEM((GB,), jnp.int32),
            pltpu.VMEM((GB,), jnp.int32),
            pltpu.VMEM((GB,), jnp.int32),
            pltpu.VMEM((GB,), jnp.float32),
            pltpu.VMEM((GB,), jnp.float32),
            pltpu.VMEM((GB,), jnp.float32),
            pltpu.VMEM((GB,), jnp.float32),
            pltpu.VMEM((GB,), jnp.int32),
            pltpu.VMEM((GB,), jnp.int32),
            pltpu.VMEM((GB,), jnp.int32),
            pltpu.VMEM((GB,), jnp.int32),
            pltpu.VMEM((PB,), jnp.float32),
            pltpu.VMEM((PB,), jnp.float32),
            pltpu.SemaphoreType.DMA,
            pltpu.SemaphoreType.DMA,
            pltpu.SemaphoreType.DMA,
            pltpu.SemaphoreType.DMA,
        ],
    )(coords_f, tpk, lvlc.reshape(N_LEVELS * NLC * 16), swb)
    enc0 = enc0f.reshape(N_LEVELS, N_PTS)
    enc1 = enc1f.reshape(N_LEVELS, N_PTS)

    blk = 2048
    out = pl.pallas_call(
        _mlp_body,
        grid=(N_PTS // blk,),
        in_specs=[
            pl.BlockSpec((N_LEVELS, blk), lambda i: (0, i)),
            pl.BlockSpec((N_LEVELS, blk), lambda i: (0, i)),
            pl.BlockSpec((N_LEVELS, 64), lambda i: (0, 0)),
            pl.BlockSpec((N_LEVELS, 64), lambda i: (0, 0)),
            pl.BlockSpec((64, 64), lambda i: (0, 0)),
            pl.BlockSpec((64, 3), lambda i: (0, 0)),
        ],
        out_specs=pl.BlockSpec((blk, 3), lambda i: (i, 0)),
        out_shape=jax.ShapeDtypeStruct((N_PTS, 3), jnp.float32),
    )(enc0, enc1, w0e, w0o, W1, W2)

    return (out, gmm)
